# baseline (speedup 1.0000x reference)
import functools

import jax
import jax.numpy as jnp
from jax import lax
from jax.experimental import pallas as pl
from jax.experimental.pallas import tpu as pltpu
from jax.experimental.pallas import tpu_sc as plsc

BATCH = 16384
NROWS = 100000
D = 32
NC = 2
NS = 16
NW = NC * NS
B_PER_W = BATCH // NW


def _tc_transpose(tab_t):
  blk = 2048

  def body(x_ref, o_ref):
    o_ref[...] = x_ref[...].T

  return pl.pallas_call(
      body,
      grid=(pl.cdiv(NROWS, blk),),
      in_specs=[pl.BlockSpec((D, blk), lambda i: (0, i))],
      out_specs=pl.BlockSpec((blk, D), lambda i: (i, 0)),
      out_shape=jax.ShapeDtypeStruct((NROWS, D), jnp.float32),
  )(tab_t)


def _sc_gather2(tab_a, tab_b, idx):
  mesh = plsc.VectorSubcoreMesh(core_axis_name="c", subcore_axis_name="s")
  out = jax.ShapeDtypeStruct((BATCH, D), jnp.float32)

  @functools.partial(
      pl.kernel,
      mesh=mesh,
      out_type=[out, out],
      compiler_params=pltpu.CompilerParams(use_tc_tiling_on_sc=False),
      scratch_types=[
          pltpu.VMEM((B_PER_W,), jnp.int32),
          pltpu.VMEM((B_PER_W, D), jnp.float32),
          pltpu.VMEM((B_PER_W, D), jnp.float32),
          pltpu.SemaphoreType.DMA,
          pltpu.SemaphoreType.DMA,
      ],
  )
  def k(ta_hbm, tb_hbm, i_hbm, oa, ob, i_v, ra, rb, gsem, osem):
    wid = lax.axis_index("s") * NC + lax.axis_index("c")
    base = wid * B_PER_W
    pltpu.sync_copy(i_hbm.at[pl.ds(base, B_PER_W)], i_v)
    ca = pltpu.async_copy(ta_hbm.at[i_v], ra, gsem)
    cb = pltpu.async_copy(tb_hbm.at[i_v], rb, gsem)
    ca.wait()
    wa = pltpu.async_copy(ra, oa.at[pl.ds(base, B_PER_W)], osem)
    cb.wait()
    wb = pltpu.async_copy(rb, ob.at[pl.ds(base, B_PER_W)], osem)
    wa.wait()
    wb.wait()

  return k(tab_a, tab_b, idx)


def _tc_mlp(gu_mlp, gi_mlp, gu_mf, gi_mf, W1a, W1b, b1, W2t, b2, W3t, b3,
            wo_mlp, wo_mf, bo):
  blk = 4096
  grid = (BATCH // blk,)

  def body(u_ref, i_ref, umf_ref, imf_ref, w1a_ref, w1b_ref, b1_ref,
           w2_ref, b2_ref, w3_ref, b3_ref, womlp_ref, womf_ref, bo_ref,
           o_ref):
    u = u_ref[...]
    it = i_ref[...]
    h = jnp.dot(u, w1a_ref[...], preferred_element_type=jnp.float32)
    h += jnp.dot(it, w1b_ref[...], preferred_element_type=jnp.float32)
    h = jnp.maximum(h + b1_ref[...], 0.0)
    h = jnp.dot(h, w2_ref[...], preferred_element_type=jnp.float32)
    h = jnp.maximum(h + b2_ref[...], 0.0)
    h = jnp.dot(h, w3_ref[...], preferred_element_type=jnp.float32)
    h = jnp.maximum(h + b3_ref[...], 0.0)
    mf = umf_ref[...] * imf_ref[...]
    logit = jnp.dot(h, womlp_ref[...], preferred_element_type=jnp.float32)
    logit += jnp.dot(mf, womf_ref[...], preferred_element_type=jnp.float32)
    o_ref[...] = jax.nn.sigmoid(logit + bo_ref[...])

  rows = pl.BlockSpec((blk, D), lambda i: (i, 0))
  full = lambda s: pl.BlockSpec(s, lambda i: tuple(0 for _ in s))
  return pl.pallas_call(
      body,
      grid=grid,
      in_specs=[
          rows, rows, rows, rows,
          full((D, D)), full((D, D)), full((1, D)),
          full((D, 16)), full((1, 16)),
          full((16, 8)), full((1, 8)),
          full((8, 1)), full((D, 1)), full((1, 1)),
      ],
      out_specs=pl.BlockSpec((blk, 1), lambda i: (i, 0)),
      out_shape=jax.ShapeDtypeStruct((BATCH, 1), jnp.float32),
  )(gu_mlp, gi_mlp, gu_mf, gi_mf, W1a, W1b, b1, W2t, b2, W3t, b3,
    wo_mlp, wo_mf, bo)


def kernel(user_indices, item_indices, emb_user_mlp, emb_item_mlp,
           emb_user_mf, emb_item_mf, W1, b1, W2, b2, W3, b3, Wo, bo):
  uidx = user_indices.astype(jnp.int32)
  iidx = item_indices.astype(jnp.int32)

  u_mlp_rm = _tc_transpose(emb_user_mlp.T)
  u_mf_rm = _tc_transpose(emb_user_mf.T)
  gu_mlp, gu_mf = _sc_gather2(u_mlp_rm, u_mf_rm, uidx)

  i_mlp_rm = _tc_transpose(emb_item_mlp.T)
  i_mf_rm = _tc_transpose(emb_item_mf.T)
  gi_mlp, gi_mf = _sc_gather2(i_mlp_rm, i_mf_rm, iidx)

  W1a = W1[:, :D].T
  W1b = W1[:, D:].T
  W2t = W2.T
  W3t = W3.T
  wo_mlp = Wo[:, :8].T
  wo_mf = Wo[:, 8:].T

  out = _tc_mlp(gu_mlp, gi_mlp, gu_mf, gi_mf,
                W1a, W1b, b1.reshape(1, -1), W2t, b2.reshape(1, -1),
                W3t, b3.reshape(1, -1), wo_mlp, wo_mf, bo.reshape(1, 1))
  return out.reshape(BATCH)


# speedup vs baseline: 2.5367x; 2.5367x over previous
import functools

import jax
import jax.numpy as jnp
from jax import lax
from jax.experimental import pallas as pl
from jax.experimental import pallas as pl
from jax.experimental.pallas import tpu as pltpu
from jax.experimental.pallas import tpu_sc as plsc

BATCH = 16384
NROWS = 100000
D = 32
NC = 2
NS = 16
NW = NC * NS
B_PER_W = BATCH // NW


def _dot_t(x, w):
  return lax.dot_general(x, w, (((0,), (0,)), ((), ())),
                         preferred_element_type=jnp.float32)


def _tc_pack(ut_mlp, it_mlp, ut_mf, it_mf, W1a, W1b):
  blk = 4096

  def body(xu_ref, xi_ref, xuf_ref, xif_ref, w1a_ref, w1b_ref, o_ref):
    eye = jnp.where(
        lax.broadcasted_iota(jnp.int32, (D, D), 0)
        == lax.broadcasted_iota(jnp.int32, (D, D), 1), 1.0, 0.0)
    o_ref[:, 0:D] = _dot_t(xu_ref[...], w1a_ref[...])
    o_ref[:, D:2 * D] = _dot_t(xi_ref[...], w1b_ref[...])
    o_ref[:, 2 * D:3 * D] = _dot_t(xuf_ref[...], eye)
    o_ref[:, 3 * D:] = _dot_t(xif_ref[...], eye)

  cols = pl.BlockSpec((D, blk), lambda i: (0, i))
  full = lambda s: pl.BlockSpec(s, lambda i: tuple(0 for _ in s))
  return pl.pallas_call(
      body,
      grid=(pl.cdiv(NROWS, blk),),
      in_specs=[cols, cols, cols, cols, full((D, D)), full((D, D))],
      out_specs=pl.BlockSpec((blk, 4 * D), lambda i: (i, 0)),
      out_shape=jax.ShapeDtypeStruct((NROWS, 4 * D), jnp.float32),
  )(ut_mlp, it_mlp, ut_mf, it_mf, W1a, W1b)


def _sc_gather(tab, uidx, iidx):
  mesh = plsc.VectorSubcoreMesh(core_axis_name="c", subcore_axis_name="s")
  out = jax.ShapeDtypeStruct((BATCH, 4 * D), jnp.float32)
  ch = 128
  nch = B_PER_W // ch
  buf = pltpu.VMEM((ch, 4 * D), jnp.float32)

  @functools.partial(
      pl.kernel,
      mesh=mesh,
      out_type=[out, out],
      compiler_params=pltpu.CompilerParams(use_tc_tiling_on_sc=True),
      scratch_types=[
          pltpu.VMEM((B_PER_W,), jnp.int32),
          pltpu.VMEM((B_PER_W,), jnp.int32),
          buf, buf,
          buf, buf,
          pltpu.SemaphoreType.DMA,
          pltpu.SemaphoreType.DMA,
          pltpu.SemaphoreType.DMA,
          pltpu.SemaphoreType.DMA,
      ],
  )
  def k(tab_hbm, ui_hbm, ii_hbm, ou, oi, ui_v, ii_v,
        bu0, bu1, bi0, bi1, gsem0, gsem1, osem0, osem1):
    wid = lax.axis_index("s") * NC + lax.axis_index("c")
    base = wid * B_PER_W
    pltpu.sync_copy(ui_hbm.at[pl.ds(base, B_PER_W)], ui_v)
    pltpu.sync_copy(ii_hbm.at[pl.ds(base, B_PER_W)], ii_v)
    bufs = [(bu0, bi0), (bu1, bi1)]
    idxs = (ui_v, ii_v)
    outs = (ou, oi)
    gsems = [gsem0, gsem1]
    osems = [osem0, osem1]

    def fire_gathers(c):
      s = c % 2
      off = c * ch
      return [
          pltpu.async_copy(tab_hbm.at[idxs[t].at[pl.ds(off, ch)]],
                           bufs[s][t], gsems[s])
          for t in range(2)
      ]

    def fire_writes(c):
      s = c % 2
      off = base + c * ch
      return [
          pltpu.async_copy(bufs[s][t], outs[t].at[pl.ds(off, ch)], osems[s])
          for t in range(2)
      ]

    pend_g = {0: fire_gathers(0)}
    pend_w = {}
    for c in range(nch):
      if c + 1 < nch:
        if c - 1 >= 0:
          for w in pend_w.pop(c - 1):
            w.wait()
        pend_g[c + 1] = fire_gathers(c + 1)
      for g in pend_g.pop(c):
        g.wait()
      pend_w[c] = fire_writes(c)
    for c in list(pend_w):
      for w in pend_w.pop(c):
        w.wait()

  return k(tab, uidx, iidx)


def _tc_mlp(gu, gi, b1, W2t, b2, W3t, b3, wo_mlp, wo_mf, bo):
  blk = 4096
  grid = (BATCH // blk,)

  def body(u_ref, i_ref, b1_ref, w2_ref, b2_ref, w3_ref, b3_ref,
           womlp_ref, womf_ref, bo_ref, o_ref):
    u = u_ref[...]
    it = i_ref[...]
    h = jnp.maximum(u[:, 0:D] + it[:, D:2 * D] + b1_ref[...], 0.0)
    h = jnp.dot(h, w2_ref[...], preferred_element_type=jnp.float32)
    h = jnp.maximum(h + b2_ref[...], 0.0)
    h = jnp.dot(h, w3_ref[...], preferred_element_type=jnp.float32)
    h = jnp.maximum(h + b3_ref[...], 0.0)
    mf = u[:, 2 * D:3 * D] * it[:, 3 * D:]
    logit = jnp.dot(h, womlp_ref[...], preferred_element_type=jnp.float32)
    logit += jnp.dot(mf, womf_ref[...], preferred_element_type=jnp.float32)
    o_ref[...] = jax.nn.sigmoid(logit + bo_ref[...])

  rows = pl.BlockSpec((blk, 4 * D), lambda i: (i, 0))
  full = lambda s: pl.BlockSpec(s, lambda i: tuple(0 for _ in s))
  return pl.pallas_call(
      body,
      grid=grid,
      in_specs=[
          rows, rows,
          full((1, D)),
          full((D, 16)), full((1, 16)),
          full((16, 8)), full((1, 8)),
          full((8, 1)), full((D, 1)), full((1, 1)),
      ],
      out_specs=pl.BlockSpec((blk, 1), lambda i: (i, 0)),
      out_shape=jax.ShapeDtypeStruct((BATCH, 1), jnp.float32),
  )(gu, gi, b1, W2t, b2, W3t, b3, wo_mlp, wo_mf, bo)


def kernel(user_indices, item_indices, emb_user_mlp, emb_item_mlp,
           emb_user_mf, emb_item_mf, W1, b1, W2, b2, W3, b3, Wo, bo):
  uidx = user_indices.astype(jnp.int32)
  iidx = item_indices.astype(jnp.int32)

  W1a = W1[:, :D].T
  W1b = W1[:, D:].T
  W2t = W2.T
  W3t = W3.T
  wo_mlp = Wo[:, :8].T
  wo_mf = Wo[:, 8:].T

  tab = _tc_pack(emb_user_mlp.T, emb_item_mlp.T, emb_user_mf.T,
                 emb_item_mf.T, W1a, W1b)
  gu, gi = _sc_gather(tab, uidx, iidx)
  out = _tc_mlp(gu, gi, b1.reshape(1, -1), W2t, b2.reshape(1, -1),
                W3t, b3.reshape(1, -1), wo_mlp, wo_mf, bo.reshape(1, 1))
  return out.reshape(BATCH)


# speedup vs baseline: 4.2233x; 1.6649x over previous
import functools

import jax
import jax.numpy as jnp
from jax import lax
from jax.experimental import pallas as pl
from jax.experimental import pallas as pl
from jax.experimental.pallas import tpu as pltpu
from jax.experimental.pallas import tpu_sc as plsc

BATCH = 16384
NROWS = 100000
D = 32
NC = 2
NS = 16
NW = NC * NS
B_PER_W = BATCH // NW


def _tc_pack(ut_mlp, it_mlp, ut_mf, it_mf, BD):
  blk = 4096

  def body(xu_ref, xi_ref, xuf_ref, xif_ref, bd_ref, o_ref):
    x = jnp.concatenate(
        [xu_ref[...], xi_ref[...], xuf_ref[...], xif_ref[...]], axis=0)
    o_ref[...] = lax.dot_general(x, bd_ref[...], (((0,), (0,)), ((), ())),
                                 preferred_element_type=jnp.float32)

  cols = pl.BlockSpec((D, blk), lambda i: (0, i))
  full = lambda s: pl.BlockSpec(s, lambda i: tuple(0 for _ in s))
  return pl.pallas_call(
      body,
      grid=(pl.cdiv(NROWS, blk),),
      in_specs=[cols, cols, cols, cols, full((4 * D, 4 * D))],
      out_specs=pl.BlockSpec((blk, 4 * D), lambda i: (i, 0)),
      out_shape=jax.ShapeDtypeStruct((NROWS, 4 * D), jnp.float32),
  )(ut_mlp, it_mlp, ut_mf, it_mf, BD)


def _sc_gather(tab, uidx, iidx):
  mesh = plsc.VectorSubcoreMesh(core_axis_name="c", subcore_axis_name="s")
  out = jax.ShapeDtypeStruct((BATCH, 4 * D), jnp.float32)
  ch = 128
  nch = B_PER_W // ch
  buf = pltpu.VMEM((ch, 4 * D), jnp.float32)

  @functools.partial(
      pl.kernel,
      mesh=mesh,
      out_type=[out, out],
      compiler_params=pltpu.CompilerParams(use_tc_tiling_on_sc=True),
      scratch_types=[
          pltpu.VMEM((B_PER_W,), jnp.int32),
          pltpu.VMEM((B_PER_W,), jnp.int32),
          buf, buf,
          buf, buf,
          pltpu.SemaphoreType.DMA,
          pltpu.SemaphoreType.DMA,
          pltpu.SemaphoreType.DMA,
          pltpu.SemaphoreType.DMA,
      ],
  )
  def k(tab_hbm, ui_hbm, ii_hbm, ou, oi, ui_v, ii_v,
        bu0, bu1, bi0, bi1, gsem0, gsem1, osem0, osem1):
    wid = lax.axis_index("s") * NC + lax.axis_index("c")
    base = wid * B_PER_W
    pltpu.sync_copy(ui_hbm.at[pl.ds(base, B_PER_W)], ui_v)
    pltpu.sync_copy(ii_hbm.at[pl.ds(base, B_PER_W)], ii_v)
    bufs = [(bu0, bi0), (bu1, bi1)]
    idxs = (ui_v, ii_v)
    outs = (ou, oi)
    gsems = [gsem0, gsem1]
    osems = [osem0, osem1]

    def fire_gathers(c):
      s = c % 2
      off = c * ch
      return [
          pltpu.async_copy(tab_hbm.at[idxs[t].at[pl.ds(off, ch)]],
                           bufs[s][t], gsems[s])
          for t in range(2)
      ]

    def fire_writes(c):
      s = c % 2
      off = base + c * ch
      return [
          pltpu.async_copy(bufs[s][t], outs[t].at[pl.ds(off, ch)], osems[s])
          for t in range(2)
      ]

    pend_g = {0: fire_gathers(0)}
    pend_w = {}
    for c in range(nch):
      if c + 1 < nch:
        if c - 1 >= 0:
          for w in pend_w.pop(c - 1):
            w.wait()
        pend_g[c + 1] = fire_gathers(c + 1)
      for g in pend_g.pop(c):
        g.wait()
      pend_w[c] = fire_writes(c)
    for c in list(pend_w):
      for w in pend_w.pop(c):
        w.wait()

  return k(tab, uidx, iidx)


def _tc_mlp(gu, gi, b1, W2t, b2, W3t, b3, wo_mlp, wo_mf, bo):
  blk = 4096
  grid = (BATCH // blk,)

  def body(u_ref, i_ref, b1_ref, w2_ref, b2_ref, w3_ref, b3_ref,
           womlp_ref, womf_ref, bo_ref, o_ref):
    u = u_ref[...]
    it = i_ref[...]
    h = jnp.maximum(u[:, 0:D] + it[:, D:2 * D] + b1_ref[...], 0.0)
    h = jnp.dot(h, w2_ref[...], preferred_element_type=jnp.float32)
    h = jnp.maximum(h + b2_ref[...], 0.0)
    h = jnp.dot(h, w3_ref[...], preferred_element_type=jnp.float32)
    h = jnp.maximum(h + b3_ref[...], 0.0)
    mf = u[:, 2 * D:3 * D] * it[:, 3 * D:]
    logit = jnp.dot(h, womlp_ref[...], preferred_element_type=jnp.float32)
    logit += jnp.dot(mf, womf_ref[...], preferred_element_type=jnp.float32)
    o_ref[...] = jax.nn.sigmoid(logit + bo_ref[...])

  rows = pl.BlockSpec((blk, 4 * D), lambda i: (i, 0))
  full = lambda s: pl.BlockSpec(s, lambda i: tuple(0 for _ in s))
  return pl.pallas_call(
      body,
      grid=grid,
      in_specs=[
          rows, rows,
          full((1, D)),
          full((D, 16)), full((1, 16)),
          full((16, 8)), full((1, 8)),
          full((8, 1)), full((D, 1)), full((1, 1)),
      ],
      out_specs=pl.BlockSpec((blk, 1), lambda i: (i, 0)),
      out_shape=jax.ShapeDtypeStruct((BATCH, 1), jnp.float32),
  )(gu, gi, b1, W2t, b2, W3t, b3, wo_mlp, wo_mf, bo)


def kernel(user_indices, item_indices, emb_user_mlp, emb_item_mlp,
           emb_user_mf, emb_item_mf, W1, b1, W2, b2, W3, b3, Wo, bo):
  uidx = user_indices.astype(jnp.int32)
  iidx = item_indices.astype(jnp.int32)

  W1a = W1[:, :D].T
  W1b = W1[:, D:].T
  W2t = W2.T
  W3t = W3.T
  wo_mlp = Wo[:, :8].T
  wo_mf = Wo[:, 8:].T

  eye = jnp.eye(D, dtype=jnp.float32)
  zero = jnp.zeros((D, D), jnp.float32)
  BD = jnp.block([
      [W1a, zero, zero, zero],
      [zero, W1b, zero, zero],
      [zero, zero, eye, zero],
      [zero, zero, zero, eye],
  ])

  tab = _tc_pack(emb_user_mlp.T, emb_item_mlp.T, emb_user_mf.T,
                 emb_item_mf.T, BD)
  gu, gi = _sc_gather(tab, uidx, iidx)
  out = _tc_mlp(gu, gi, b1.reshape(1, -1), W2t, b2.reshape(1, -1),
                W3t, b3.reshape(1, -1), wo_mlp, wo_mf, bo.reshape(1, 1))
  return out.reshape(BATCH)


# speedup vs baseline: 4.5733x; 1.0829x over previous
import functools

import jax
import jax.numpy as jnp
from jax import lax
from jax.experimental import pallas as pl
from jax.experimental import pallas as pl
from jax.experimental.pallas import tpu as pltpu
from jax.experimental.pallas import tpu_sc as plsc

BATCH = 16384
NROWS = 100000
D = 32
NC = 2
NS = 16
NW = NC * NS
B_PER_W = BATCH // NW


def _tc_pack(ut_mlp, it_mlp, ut_mf, it_mf, BD):
  blk = 8192

  def body(xu_ref, xi_ref, xuf_ref, xif_ref, bd_ref, o_ref):
    x = jnp.concatenate(
        [xu_ref[...], xi_ref[...], xuf_ref[...], xif_ref[...]], axis=0)
    o_ref[...] = lax.dot_general(x, bd_ref[...], (((0,), (0,)), ((), ())),
                                 preferred_element_type=jnp.float32)

  cols = pl.BlockSpec((D, blk), lambda i: (0, i))
  full = lambda s: pl.BlockSpec(s, lambda i: tuple(0 for _ in s))
  return pl.pallas_call(
      body,
      grid=(pl.cdiv(NROWS, blk),),
      in_specs=[cols, cols, cols, cols, full((4 * D, 4 * D))],
      out_specs=pl.BlockSpec((blk, 4 * D), lambda i: (i, 0)),
      out_shape=jax.ShapeDtypeStruct((NROWS, 4 * D), jnp.float32),
      compiler_params=pltpu.CompilerParams(
          dimension_semantics=("parallel",)),
  )(ut_mlp, it_mlp, ut_mf, it_mf, BD)


def _sc_gather(tab, uidx, iidx):
  mesh = plsc.VectorSubcoreMesh(core_axis_name="c", subcore_axis_name="s")
  out = jax.ShapeDtypeStruct((BATCH, 4 * D), jnp.float32)
  ch = 128
  nch = B_PER_W // ch
  buf = pltpu.VMEM((ch, 4 * D), jnp.float32)

  @functools.partial(
      pl.kernel,
      mesh=mesh,
      out_type=[out, out],
      compiler_params=pltpu.CompilerParams(use_tc_tiling_on_sc=True),
      scratch_types=[
          pltpu.VMEM((B_PER_W,), jnp.int32),
          pltpu.VMEM((B_PER_W,), jnp.int32),
          buf, buf,
          buf, buf,
          pltpu.SemaphoreType.DMA,
          pltpu.SemaphoreType.DMA,
          pltpu.SemaphoreType.DMA,
          pltpu.SemaphoreType.DMA,
      ],
  )
  def k(tab_hbm, ui_hbm, ii_hbm, ou, oi, ui_v, ii_v,
        bu0, bu1, bi0, bi1, gsem0, gsem1, osem0, osem1):
    wid = lax.axis_index("s") * NC + lax.axis_index("c")
    base = wid * B_PER_W
    pltpu.sync_copy(ui_hbm.at[pl.ds(base, B_PER_W)], ui_v)
    pltpu.sync_copy(ii_hbm.at[pl.ds(base, B_PER_W)], ii_v)
    bufs = [(bu0, bi0), (bu1, bi1)]
    idxs = (ui_v, ii_v)
    outs = (ou, oi)
    gsems = [gsem0, gsem1]
    osems = [osem0, osem1]

    def fire_gathers(c):
      s = c % 2
      off = c * ch
      return [
          pltpu.async_copy(tab_hbm.at[idxs[t].at[pl.ds(off, ch)]],
                           bufs[s][t], gsems[s])
          for t in range(2)
      ]

    def fire_writes(c):
      s = c % 2
      off = base + c * ch
      return [
          pltpu.async_copy(bufs[s][t], outs[t].at[pl.ds(off, ch)], osems[s])
          for t in range(2)
      ]

    pend_g = {0: fire_gathers(0)}
    pend_w = {}
    for c in range(nch):
      if c + 1 < nch:
        if c - 1 >= 0:
          for w in pend_w.pop(c - 1):
            w.wait()
        pend_g[c + 1] = fire_gathers(c + 1)
      for g in pend_g.pop(c):
        g.wait()
      pend_w[c] = fire_writes(c)
    for c in list(pend_w):
      for w in pend_w.pop(c):
        w.wait()

  return k(tab, uidx, iidx)


def _tc_mlp(gu, gi, b1, W2t, b2, W3t, b3, wo_mlp, wo_mf, bo):
  blk = 8192
  grid = (BATCH // blk,)

  def body(u_ref, i_ref, b1_ref, w2_ref, b2_ref, w3_ref, b3_ref,
           womlp_ref, womf_ref, bo_ref, o_ref):
    u = u_ref[...]
    it = i_ref[...]
    h = jnp.maximum(u[:, 0:D] + it[:, D:2 * D] + b1_ref[...], 0.0)
    h = jnp.dot(h, w2_ref[...], preferred_element_type=jnp.float32)
    h = jnp.maximum(h + b2_ref[...], 0.0)
    h = jnp.dot(h, w3_ref[...], preferred_element_type=jnp.float32)
    h = jnp.maximum(h + b3_ref[...], 0.0)
    mf = u[:, 2 * D:3 * D] * it[:, 3 * D:]
    logit = jnp.dot(h, womlp_ref[...], preferred_element_type=jnp.float32)
    logit += jnp.dot(mf, womf_ref[...], preferred_element_type=jnp.float32)
    o_ref[...] = jax.nn.sigmoid(logit + bo_ref[...])[:, 0]

  rows = pl.BlockSpec((blk, 4 * D), lambda i: (i, 0))
  full = lambda s: pl.BlockSpec(s, lambda i: tuple(0 for _ in s))
  return pl.pallas_call(
      body,
      grid=grid,
      in_specs=[
          rows, rows,
          full((1, D)),
          full((D, 16)), full((1, 16)),
          full((16, 8)), full((1, 8)),
          full((8, 1)), full((D, 1)), full((1, 1)),
      ],
      out_specs=pl.BlockSpec((blk,), lambda i: (i,)),
      out_shape=jax.ShapeDtypeStruct((BATCH,), jnp.float32),
      compiler_params=pltpu.CompilerParams(
          dimension_semantics=("parallel",)),
  )(gu, gi, b1, W2t, b2, W3t, b3, wo_mlp, wo_mf, bo)


def kernel(user_indices, item_indices, emb_user_mlp, emb_item_mlp,
           emb_user_mf, emb_item_mf, W1, b1, W2, b2, W3, b3, Wo, bo):
  uidx = user_indices.astype(jnp.int32)
  iidx = item_indices.astype(jnp.int32)

  W1a = W1[:, :D].T
  W1b = W1[:, D:].T
  W2t = W2.T
  W3t = W3.T
  wo_mlp = Wo[:, :8].T
  wo_mf = Wo[:, 8:].T

  eye = jnp.eye(D, dtype=jnp.float32)
  zero = jnp.zeros((D, D), jnp.float32)
  BD = jnp.block([
      [W1a, zero, zero, zero],
      [zero, W1b, zero, zero],
      [zero, zero, eye, zero],
      [zero, zero, zero, eye],
  ])

  tab = _tc_pack(emb_user_mlp.T, emb_item_mlp.T, emb_user_mf.T,
                 emb_item_mf.T, BD)
  gu, gi = _sc_gather(tab, uidx, iidx)
  return _tc_mlp(gu, gi, b1.reshape(1, -1), W2t, b2.reshape(1, -1),
                 W3t, b3.reshape(1, -1), wo_mlp, wo_mf, bo.reshape(1, 1))


# speedup vs baseline: 5.0203x; 1.0977x over previous
import functools

import jax
import jax.numpy as jnp
from jax import lax
from jax.experimental import pallas as pl
from jax.experimental import pallas as pl
from jax.experimental.pallas import tpu as pltpu
from jax.experimental.pallas import tpu_sc as plsc

BATCH = 16384
NROWS = 100000
D = 32
NC = 2
NS = 16
NW = NC * NS
B_PER_W = BATCH // NW


def _tc_pack(ut_mlp, it_mlp, ut_mf, it_mf, BD):
  blk = 8192

  def body(xu_ref, xi_ref, xuf_ref, xif_ref, bd_ref, o_ref):
    x = jnp.concatenate(
        [xu_ref[...], xi_ref[...], xuf_ref[...], xif_ref[...]], axis=0)
    o_ref[...] = lax.dot_general(x, bd_ref[...], (((0,), (0,)), ((), ())),
                                 preferred_element_type=jnp.float32)

  cols = pl.BlockSpec((D, blk), lambda i: (0, i))
  full = lambda s: pl.BlockSpec(s, lambda i: tuple(0 for _ in s))
  return pl.pallas_call(
      body,
      grid=(pl.cdiv(NROWS, blk),),
      in_specs=[cols, cols, cols, cols, full((4 * D, 4 * D))],
      out_specs=pl.BlockSpec((blk, 4 * D), lambda i: (i, 0)),
      out_shape=jax.ShapeDtypeStruct((NROWS, 4 * D), jnp.float32),
      compiler_params=pltpu.CompilerParams(
          dimension_semantics=("parallel",)),
  )(ut_mlp, it_mlp, ut_mf, it_mf, BD)


def _sc_gather(tab, uidx, iidx):
  mesh = plsc.VectorSubcoreMesh(core_axis_name="c", subcore_axis_name="s")
  out = jax.ShapeDtypeStruct((BATCH, 4 * D), jnp.float32)
  ch = 128
  nch = B_PER_W // ch
  buf = pltpu.VMEM((ch, 4 * D), jnp.float32)

  @functools.partial(
      pl.kernel,
      mesh=mesh,
      out_type=[out, out],
      compiler_params=pltpu.CompilerParams(use_tc_tiling_on_sc=True),
      scratch_types=[
          pltpu.VMEM((B_PER_W,), jnp.int32),
          pltpu.VMEM((B_PER_W,), jnp.int32),
          buf, buf,
          buf, buf,
          pltpu.SemaphoreType.DMA,
          pltpu.SemaphoreType.DMA,
          pltpu.SemaphoreType.DMA,
          pltpu.SemaphoreType.DMA,
      ],
  )
  def k(tab_hbm, ui_hbm, ii_hbm, ou, oi, ui_v, ii_v,
        bu0, bu1, bi0, bi1, gsem0, gsem1, osem0, osem1):
    wid = lax.axis_index("s") * NC + lax.axis_index("c")
    base = wid * B_PER_W
    pltpu.sync_copy(ui_hbm.at[pl.ds(base, B_PER_W)], ui_v)
    pltpu.sync_copy(ii_hbm.at[pl.ds(base, B_PER_W)], ii_v)
    bufs = [(bu0, bi0), (bu1, bi1)]
    idxs = (ui_v, ii_v)
    outs = (ou, oi)
    gsems = [gsem0, gsem1]
    osems = [osem0, osem1]

    def fire_gathers(c):
      s = c % 2
      off = c * ch
      return [
          pltpu.async_copy(tab_hbm.at[idxs[t].at[pl.ds(off, ch)]],
                           bufs[s][t], gsems[s])
          for t in range(2)
      ]

    def fire_writes(c):
      s = c % 2
      off = base + c * ch
      return [
          pltpu.async_copy(bufs[s][t], outs[t].at[pl.ds(off, ch)], osems[s])
          for t in range(2)
      ]

    pend_g = {0: fire_gathers(0)}
    pend_w = {}
    for c in range(nch):
      if c + 1 < nch:
        if c - 1 >= 0:
          for w in pend_w.pop(c - 1):
            w.wait()
        pend_g[c + 1] = fire_gathers(c + 1)
      for g in pend_g.pop(c):
        g.wait()
      pend_w[c] = fire_writes(c)
    for c in list(pend_w):
      for w in pend_w.pop(c):
        w.wait()

  return k(tab, uidx, iidx)


def _tc_mlp(gu, gi, b1, W2t, b2, W3t, b3, wo_mlp, wo_mf, bo):
  blk = 4096
  grid = (BATCH // blk,)

  def body(u_ref, i_ref, b1_ref, w2_ref, b2_ref, w3_ref, b3_ref,
           womlp_ref, womf_ref, bo_ref, o_ref):
    u = u_ref[...]
    it = i_ref[...]
    h = jnp.maximum(u[:, 0:D] + it[:, D:2 * D] + b1_ref[...], 0.0)
    h = jnp.dot(h, w2_ref[...], preferred_element_type=jnp.float32)
    h = jnp.maximum(h + b2_ref[...], 0.0)
    h = jnp.dot(h, w3_ref[...], preferred_element_type=jnp.float32)
    h = jnp.maximum(h + b3_ref[...], 0.0)
    mf = u[:, 2 * D:3 * D] * it[:, 3 * D:]
    logit_t = lax.dot_general(womlp_ref[...], h, (((1,), (1,)), ((), ())),
                              preferred_element_type=jnp.float32)
    logit_t += lax.dot_general(womf_ref[...], mf, (((1,), (1,)), ((), ())),
                               preferred_element_type=jnp.float32)
    o_ref[...] = jax.nn.sigmoid(logit_t + bo_ref[...])[0]

  rows = pl.BlockSpec((blk, 4 * D), lambda i: (i, 0))
  full = lambda s: pl.BlockSpec(s, lambda i: tuple(0 for _ in s))
  return pl.pallas_call(
      body,
      grid=grid,
      in_specs=[
          rows, rows,
          full((1, D)),
          full((D, 16)), full((1, 16)),
          full((16, 8)), full((1, 8)),
          full((1, 8)), full((1, D)), full((1, 1)),
      ],
      out_specs=pl.BlockSpec((blk,), lambda i: (i,)),
      out_shape=jax.ShapeDtypeStruct((BATCH,), jnp.float32),
      compiler_params=pltpu.CompilerParams(
          dimension_semantics=("parallel",)),
  )(gu, gi, b1, W2t, b2, W3t, b3, wo_mlp, wo_mf, bo)


def kernel(user_indices, item_indices, emb_user_mlp, emb_item_mlp,
           emb_user_mf, emb_item_mf, W1, b1, W2, b2, W3, b3, Wo, bo):
  uidx = user_indices.astype(jnp.int32)
  iidx = item_indices.astype(jnp.int32)

  W1a = W1[:, :D].T
  W1b = W1[:, D:].T
  W2t = W2.T
  W3t = W3.T
  wo_mlp = Wo[:, :8]
  wo_mf = Wo[:, 8:]

  eye = jnp.eye(D, dtype=jnp.float32)
  zero = jnp.zeros((D, D), jnp.float32)
  BD = jnp.block([
      [W1a, zero, zero, zero],
      [zero, W1b, zero, zero],
      [zero, zero, eye, zero],
      [zero, zero, zero, eye],
  ])

  tab = _tc_pack(emb_user_mlp.T, emb_item_mlp.T, emb_user_mf.T,
                 emb_item_mf.T, BD)
  gu, gi = _sc_gather(tab, uidx, iidx)
  return _tc_mlp(gu, gi, b1.reshape(1, -1), W2t, b2.reshape(1, -1),
                 W3t, b3.reshape(1, -1), wo_mlp, wo_mf, bo.reshape(1, 1))


# speedup vs baseline: 5.0641x; 1.0087x over previous
import functools

import jax
import jax.numpy as jnp
from jax import lax
from jax.experimental import pallas as pl
from jax.experimental.pallas import tpu as pltpu
from jax.experimental.pallas import tpu_sc as plsc

BATCH = 16384
NROWS = 100000
D = 32
NC = 2
NS = 16
NW = NC * NS
B_PER_W = BATCH // NW


def _tc_pack(ut_mlp, it_mlp, ut_mf, it_mf, BD):
  blk = 12288

  def body(xu_ref, xi_ref, xuf_ref, xif_ref, bd_ref, o_ref):
    x = jnp.concatenate(
        [xu_ref[...], xi_ref[...], xuf_ref[...], xif_ref[...]], axis=0)
    o_ref[...] = lax.dot_general(x, bd_ref[...], (((0,), (0,)), ((), ())),
                                 preferred_element_type=jnp.float32)

  cols = pl.BlockSpec((D, blk), lambda i: (0, i))
  full = lambda s: pl.BlockSpec(s, lambda i: tuple(0 for _ in s))
  return pl.pallas_call(
      body,
      grid=(pl.cdiv(NROWS, blk),),
      in_specs=[cols, cols, cols, cols, full((4 * D, 4 * D))],
      out_specs=pl.BlockSpec((blk, 4 * D), lambda i: (i, 0)),
      out_shape=jax.ShapeDtypeStruct((NROWS, 4 * D), jnp.float32),
      compiler_params=pltpu.CompilerParams(
          dimension_semantics=("parallel",)),
  )(ut_mlp, it_mlp, ut_mf, it_mf, BD)


def _sc_gather(tab, uidx, iidx):
  mesh = plsc.VectorSubcoreMesh(core_axis_name="c", subcore_axis_name="s")
  out = jax.ShapeDtypeStruct((BATCH, 4 * D), jnp.float32)
  ch = 128
  nch = B_PER_W // ch
  buf = pltpu.VMEM((ch, 4 * D), jnp.float32)

  @functools.partial(
      pl.kernel,
      mesh=mesh,
      out_type=[out, out],
      compiler_params=pltpu.CompilerParams(use_tc_tiling_on_sc=True),
      scratch_types=[
          pltpu.VMEM((B_PER_W,), jnp.int32),
          pltpu.VMEM((B_PER_W,), jnp.int32),
          buf, buf,
          buf, buf,
          pltpu.SemaphoreType.DMA,
          pltpu.SemaphoreType.DMA,
          pltpu.SemaphoreType.DMA,
          pltpu.SemaphoreType.DMA,
      ],
  )
  def k(tab_hbm, ui_hbm, ii_hbm, ou, oi, ui_v, ii_v,
        bu0, bu1, bi0, bi1, gsem0, gsem1, osem0, osem1):
    wid = lax.axis_index("s") * NC + lax.axis_index("c")
    base = wid * B_PER_W
    pltpu.sync_copy(ui_hbm.at[pl.ds(base, B_PER_W)], ui_v)
    pltpu.sync_copy(ii_hbm.at[pl.ds(base, B_PER_W)], ii_v)
    bufs = [(bu0, bi0), (bu1, bi1)]
    idxs = (ui_v, ii_v)
    outs = (ou, oi)
    gsems = [gsem0, gsem1]
    osems = [osem0, osem1]

    def fire_gathers(c):
      s = c % 2
      off = c * ch
      return [
          pltpu.async_copy(tab_hbm.at[idxs[t].at[pl.ds(off, ch)]],
                           bufs[s][t], gsems[s])
          for t in range(2)
      ]

    def fire_writes(c):
      s = c % 2
      off = base + c * ch
      return [
          pltpu.async_copy(bufs[s][t], outs[t].at[pl.ds(off, ch)], osems[s])
          for t in range(2)
      ]

    pend_g = {0: fire_gathers(0)}
    pend_w = {}
    for c in range(nch):
      if c + 1 < nch:
        if c - 1 >= 0:
          for w in pend_w.pop(c - 1):
            w.wait()
        pend_g[c + 1] = fire_gathers(c + 1)
      for g in pend_g.pop(c):
        g.wait()
      pend_w[c] = fire_writes(c)
    for c in list(pend_w):
      for w in pend_w.pop(c):
        w.wait()

  return k(tab, uidx, iidx)


def _tc_mlp(gu, gi, b1, W2t, b2, W3t, b3, wo_mlp, wo_mf, bo):
  blk = 4096
  grid = (BATCH // blk,)

  def body(u_ref, i_ref, b1_ref, w2_ref, b2_ref, w3_ref, b3_ref,
           womlp_ref, womf_ref, bo_ref, o_ref):
    u = u_ref[...]
    it = i_ref[...]
    h = jnp.maximum(u[:, 0:D] + it[:, D:2 * D] + b1_ref[...], 0.0)
    h = jnp.dot(h, w2_ref[...], preferred_element_type=jnp.float32)
    h = jnp.maximum(h + b2_ref[...], 0.0)
    h = jnp.dot(h, w3_ref[...], preferred_element_type=jnp.float32)
    h = jnp.maximum(h + b3_ref[...], 0.0)
    mf = u[:, 2 * D:3 * D] * it[:, 3 * D:]
    logit_t = lax.dot_general(womlp_ref[...], h, (((1,), (1,)), ((), ())),
                              preferred_element_type=jnp.float32)
    logit_t += lax.dot_general(womf_ref[...], mf, (((1,), (1,)), ((), ())),
                               preferred_element_type=jnp.float32)
    o_ref[...] = jax.nn.sigmoid(logit_t + bo_ref[...])[0]

  rows = pl.BlockSpec((blk, 4 * D), lambda i: (i, 0))
  full = lambda s: pl.BlockSpec(s, lambda i: tuple(0 for _ in s))
  return pl.pallas_call(
      body,
      grid=grid,
      in_specs=[
          rows, rows,
          full((1, D)),
          full((D, 16)), full((1, 16)),
          full((16, 8)), full((1, 8)),
          full((1, 8)), full((1, D)), full((1, 1)),
      ],
      out_specs=pl.BlockSpec((blk,), lambda i: (i,)),
      out_shape=jax.ShapeDtypeStruct((BATCH,), jnp.float32),
      compiler_params=pltpu.CompilerParams(
          dimension_semantics=("parallel",)),
  )(gu, gi, b1, W2t, b2, W3t, b3, wo_mlp, wo_mf, bo)


def kernel(user_indices, item_indices, emb_user_mlp, emb_item_mlp,
           emb_user_mf, emb_item_mf, W1, b1, W2, b2, W3, b3, Wo, bo):
  uidx = user_indices.astype(jnp.int32)
  iidx = item_indices.astype(jnp.int32)

  W1a = W1[:, :D].T
  W1b = W1[:, D:].T
  W2t = W2.T
  W3t = W3.T
  wo_mlp = Wo[:, :8]
  wo_mf = Wo[:, 8:]

  eye = jnp.eye(D, dtype=jnp.float32)
  zero = jnp.zeros((D, D), jnp.float32)
  BD = jnp.block([
      [W1a, zero, zero, zero],
      [zero, W1b, zero, zero],
      [zero, zero, eye, zero],
      [zero, zero, zero, eye],
  ])

  tab = _tc_pack(emb_user_mlp.T, emb_item_mlp.T, emb_user_mf.T,
                 emb_item_mf.T, BD)
  gu, gi = _sc_gather(tab, uidx, iidx)
  return _tc_mlp(gu, gi, b1.reshape(1, -1), W2t, b2.reshape(1, -1),
                 W3t, b3.reshape(1, -1), wo_mlp, wo_mf, bo.reshape(1, 1))
